# baseline (device time: 60410 ns/iter reference)
import jax
import jax.numpy as jnp
from jax import lax
from jax.experimental import pallas as pl
from jax.experimental.pallas import tpu as pltpu

N_DEV = 4
SQ = 1024
HQ_LOCAL = 8
DH = 128
D_MODEL = 1024
D_HEADS_LOCAL = HQ_LOCAL * DH
SCALE = 0.08838834764831843
CHUNK = 128


def _store_chunk(out_ref, ring, c, val):
    base_pb = 4 * c + (2 if ring == 1 else 0)
    for j in range(2):
        pb = base_pb + j
        orig = ((pb % 4) * 4 + pb // 4) * 64
        out_ref[pl.ds(orig, 64), :] = val[j * 64:(j + 1) * 64]


def _body(x_ref, wq_hbm, k_ref, v_ref, wo_hbm, out_ref,
          wq_ref, wo_ref, xp_ref, kp_ref, vp_ref, ctx_ref, part_ref,
          commA, commB, cp_sems, semA_s, semA_r, semB_s, semB_r):
    my = lax.axis_index("i")
    left = lax.rem(my + N_DEV - 1, N_DEV)
    right = lax.rem(my + 1, N_DEV)

    cp_wq = pltpu.make_async_copy(
        wq_hbm.at[:, pl.ds(my * D_HEADS_LOCAL, D_HEADS_LOCAL)],
        wq_ref, cp_sems.at[0])
    cp_wo = pltpu.make_async_copy(
        wo_hbm.at[pl.ds(my * D_HEADS_LOCAL, D_HEADS_LOCAL), :],
        wo_ref, cp_sems.at[1])
    cp_wq.start()
    cp_wo.start()

    barrier_sem = pltpu.get_barrier_semaphore()
    for nbr in (left, right):
        pl.semaphore_signal(
            barrier_sem, inc=1,
            device_id=(nbr,), device_id_type=pl.DeviceIdType.MESH,
        )
    pl.semaphore_wait(barrier_sem, 2)

    for pb in range(16):
        dst = pl.ds(pb * 64, 64)
        src = pl.ds(((pb % 4) * 4 + pb // 4) * 64, 64)
        xp_ref[dst, :] = x_ref[src, :]
        kp_ref[dst, :] = k_ref[src, :]
        vp_ref[dst, :] = v_ref[src, :]

    cp_wq.wait()
    cp_wo.wait()

    def compute_class(cls):
        rows = pl.ds(cls * 256, 256)
        qc = jnp.dot(xp_ref[rows, :], wq_ref[:, :],
                     preferred_element_type=jnp.float32) * SCALE
        for h in range(HQ_LOCAL):
            hc = slice(h * DH, (h + 1) * DH)
            kc = kp_ref[rows, hc]
            vc = vp_ref[rows, hc]
            s = lax.dot_general(qc[:, hc], kc, (((1,), (1,)), ((), ())),
                                preferred_element_type=jnp.float32)
            m = jnp.max(s, axis=1, keepdims=True)
            w = jnp.exp(s - m)
            w = w / jnp.sum(w, axis=1, keepdims=True)
            ctx_ref[rows, hc] = jnp.dot(w, vc,
                                        preferred_element_type=jnp.float32)
        part_ref[rows, :] = jnp.dot(ctx_ref[rows, :], wo_ref[:, :],
                                    preferred_element_type=jnp.float32)

    compute_class(my)
    commA[0] = part_ref[pl.ds(my * 256, CHUNK), :]
    commB[0] = part_ref[pl.ds(my * 256 + CHUNK, CHUNK), :]

    for g in range(6):
        send_slot = g % 2
        recv_slot = (g + 1) % 2
        rdmaA = pltpu.make_async_remote_copy(
            src_ref=commA.at[send_slot], dst_ref=commA.at[recv_slot],
            send_sem=semA_s.at[send_slot], recv_sem=semA_r.at[recv_slot],
            device_id=(right,), device_id_type=pl.DeviceIdType.MESH,
        )
        rdmaB = pltpu.make_async_remote_copy(
            src_ref=commB.at[send_slot], dst_ref=commB.at[recv_slot],
            send_sem=semB_s.at[send_slot], recv_sem=semB_r.at[recv_slot],
            device_id=(left,), device_id_type=pl.DeviceIdType.MESH,
        )
        rdmaA.start()
        rdmaB.start()

        if g == 0:
            compute_class(lax.rem(my + 1, N_DEV))
            compute_class(lax.rem(my + 3, N_DEV))
        elif g == 1:
            compute_class(lax.rem(my + 2, N_DEV))

        rdmaA.wait()
        rdmaB.wait()

        if g < 3:
            cA = lax.rem(my - g - 1 + 2 * N_DEV, N_DEV)
            cB = lax.rem(my + g + 1, N_DEV)
            commA[recv_slot] = commA[recv_slot] + part_ref[
                pl.ds(cA * 256, CHUNK), :]
            commB[recv_slot] = commB[recv_slot] + part_ref[
                pl.ds(cB * 256 + CHUNK, CHUNK), :]
            if g == 2:
                _store_chunk(out_ref, 0, lax.rem(my + 1, N_DEV),
                             commA[recv_slot])
                _store_chunk(out_ref, 1, lax.rem(my + N_DEV - 1, N_DEV),
                             commB[recv_slot])
        else:
            t = g - 3
            cA = lax.rem(my - t + N_DEV, N_DEV)
            cB = lax.rem(my + t, N_DEV)
            _store_chunk(out_ref, 0, cA, commA[recv_slot])
            _store_chunk(out_ref, 1, cB, commB[recv_slot])


def kernel(x, Wq, K_ext, V_ext, Wo):
    x2 = x[0]
    k2 = K_ext[0].reshape(SQ, D_HEADS_LOCAL)
    v2 = V_ext[0].reshape(SQ, D_HEADS_LOCAL)

    out = pl.pallas_call(
        _body,
        out_shape=jax.ShapeDtypeStruct((SQ, D_MODEL), jnp.float32),
        in_specs=[
            pl.BlockSpec(memory_space=pltpu.MemorySpace.VMEM),
            pl.BlockSpec(memory_space=pltpu.MemorySpace.HBM),
            pl.BlockSpec(memory_space=pltpu.MemorySpace.VMEM),
            pl.BlockSpec(memory_space=pltpu.MemorySpace.VMEM),
            pl.BlockSpec(memory_space=pltpu.MemorySpace.HBM),
        ],
        out_specs=pl.BlockSpec(memory_space=pltpu.MemorySpace.VMEM),
        scratch_shapes=[
            pltpu.VMEM((D_MODEL, D_HEADS_LOCAL), jnp.float32),
            pltpu.VMEM((D_HEADS_LOCAL, D_MODEL), jnp.float32),
            pltpu.VMEM((SQ, D_MODEL), jnp.float32),
            pltpu.VMEM((SQ, D_HEADS_LOCAL), jnp.float32),
            pltpu.VMEM((SQ, D_HEADS_LOCAL), jnp.float32),
            pltpu.VMEM((SQ, D_HEADS_LOCAL), jnp.float32),
            pltpu.VMEM((SQ, D_MODEL), jnp.float32),
            pltpu.VMEM((2, CHUNK, D_MODEL), jnp.float32),
            pltpu.VMEM((2, CHUNK, D_MODEL), jnp.float32),
            pltpu.SemaphoreType.DMA((2,)),
            pltpu.SemaphoreType.DMA((2,)),
            pltpu.SemaphoreType.DMA((2,)),
            pltpu.SemaphoreType.DMA((2,)),
            pltpu.SemaphoreType.DMA((2,)),
        ],
        compiler_params=pltpu.CompilerParams(collective_id=0),
    )(x2, Wq, k2, v2, Wo)
    return out[None]


# device time: 43570 ns/iter; 1.3865x vs baseline; 1.3865x over previous
import jax
import jax.numpy as jnp
from jax import lax
from jax.experimental import pallas as pl
from jax.experimental.pallas import tpu as pltpu

N_DEV = 4
SQ = 1024
HQ_LOCAL = 8
DH = 128
D_MODEL = 1024
D_HEADS_LOCAL = HQ_LOCAL * DH
SCALE = 0.08838834764831843
CHUNK = 128


def _store_chunk(out_ref, ring, c, val):
    base_pb = 4 * c + (2 if ring == 1 else 0)
    val = val.astype(jnp.float32)
    for j in range(2):
        pb = base_pb + j
        orig = ((pb % 4) * 4 + pb // 4) * 64
        out_ref[pl.ds(orig, 64), :] = val[j * 64:(j + 1) * 64]


def _body(x_ref, wq_hbm, k_ref, v_ref, wo_hbm, out_ref,
          wq_ref, wo_ref, xp_ref, kp_ref, vp_ref, ctx_ref, part_ref,
          commA, commB, cp_sems, semA_s, semA_r, semB_s, semB_r):
    my = lax.axis_index("i")
    left = lax.rem(my + N_DEV - 1, N_DEV)
    right = lax.rem(my + 1, N_DEV)

    cp_wq = pltpu.make_async_copy(
        wq_hbm.at[:, pl.ds(my * D_HEADS_LOCAL, D_HEADS_LOCAL)],
        wq_ref, cp_sems.at[0])
    cp_wo = pltpu.make_async_copy(
        wo_hbm.at[pl.ds(my * D_HEADS_LOCAL, D_HEADS_LOCAL), :],
        wo_ref, cp_sems.at[1])
    cp_wq.start()
    cp_wo.start()

    barrier_sem = pltpu.get_barrier_semaphore()
    for nbr in (left, right):
        pl.semaphore_signal(
            barrier_sem, inc=1,
            device_id=(nbr,), device_id_type=pl.DeviceIdType.MESH,
        )
    pl.semaphore_wait(barrier_sem, 2)

    for pb in range(16):
        dst = pl.ds(pb * 64, 64)
        src = pl.ds(((pb % 4) * 4 + pb // 4) * 64, 64)
        xp_ref[dst, :] = x_ref[src, :]
        kp_ref[dst, :] = k_ref[src, :]
        vp_ref[dst, :] = v_ref[src, :]

    cp_wq.wait()
    cp_wo.wait()

    def compute_class(cls):
        rows = pl.ds(cls * 256, 256)
        qc = jnp.dot(xp_ref[rows, :], wq_ref[:, :],
                     preferred_element_type=jnp.float32) * SCALE
        for h in range(HQ_LOCAL):
            hc = slice(h * DH, (h + 1) * DH)
            kc = kp_ref[rows, hc]
            vc = vp_ref[rows, hc]
            s = lax.dot_general(qc[:, hc], kc, (((1,), (1,)), ((), ())),
                                preferred_element_type=jnp.float32)
            m = jnp.max(s, axis=1, keepdims=True)
            w = jnp.exp(s - m)
            w = w / jnp.sum(w, axis=1, keepdims=True)
            ctx_ref[rows, hc] = jnp.dot(w, vc,
                                        preferred_element_type=jnp.float32)
        part_ref[rows, :] = jnp.dot(ctx_ref[rows, :], wo_ref[:, :],
                                    preferred_element_type=jnp.float32)

    compute_class(my)
    commA[0] = part_ref[pl.ds(my * 256, CHUNK), :].astype(jnp.bfloat16)
    commB[0] = part_ref[pl.ds(my * 256 + CHUNK, CHUNK), :].astype(jnp.bfloat16)

    for g in range(6):
        send_slot = g % 2
        recv_slot = (g + 1) % 2
        rdmaA = pltpu.make_async_remote_copy(
            src_ref=commA.at[send_slot], dst_ref=commA.at[recv_slot],
            send_sem=semA_s.at[send_slot], recv_sem=semA_r.at[recv_slot],
            device_id=(right,), device_id_type=pl.DeviceIdType.MESH,
        )
        rdmaB = pltpu.make_async_remote_copy(
            src_ref=commB.at[send_slot], dst_ref=commB.at[recv_slot],
            send_sem=semB_s.at[send_slot], recv_sem=semB_r.at[recv_slot],
            device_id=(left,), device_id_type=pl.DeviceIdType.MESH,
        )
        rdmaA.start()
        rdmaB.start()

        if g == 0:
            compute_class(lax.rem(my + 1, N_DEV))
            compute_class(lax.rem(my + 3, N_DEV))
        elif g == 1:
            compute_class(lax.rem(my + 2, N_DEV))

        rdmaA.wait()
        rdmaB.wait()

        if g < 3:
            cA = lax.rem(my - g - 1 + 2 * N_DEV, N_DEV)
            cB = lax.rem(my + g + 1, N_DEV)
            commA[recv_slot] = (commA[recv_slot].astype(jnp.float32)
                                + part_ref[pl.ds(cA * 256, CHUNK), :]
                                ).astype(jnp.bfloat16)
            commB[recv_slot] = (commB[recv_slot].astype(jnp.float32)
                                + part_ref[pl.ds(cB * 256 + CHUNK, CHUNK), :]
                                ).astype(jnp.bfloat16)
            if g == 2:
                _store_chunk(out_ref, 0, lax.rem(my + 1, N_DEV),
                             commA[recv_slot])
                _store_chunk(out_ref, 1, lax.rem(my + N_DEV - 1, N_DEV),
                             commB[recv_slot])
        else:
            t = g - 3
            cA = lax.rem(my - t + N_DEV, N_DEV)
            cB = lax.rem(my + t, N_DEV)
            _store_chunk(out_ref, 0, cA, commA[recv_slot])
            _store_chunk(out_ref, 1, cB, commB[recv_slot])


def kernel(x, Wq, K_ext, V_ext, Wo):
    x2 = x[0]
    k2 = K_ext[0].reshape(SQ, D_HEADS_LOCAL)
    v2 = V_ext[0].reshape(SQ, D_HEADS_LOCAL)

    out = pl.pallas_call(
        _body,
        out_shape=jax.ShapeDtypeStruct((SQ, D_MODEL), jnp.float32),
        in_specs=[
            pl.BlockSpec(memory_space=pltpu.MemorySpace.VMEM),
            pl.BlockSpec(memory_space=pltpu.MemorySpace.HBM),
            pl.BlockSpec(memory_space=pltpu.MemorySpace.VMEM),
            pl.BlockSpec(memory_space=pltpu.MemorySpace.VMEM),
            pl.BlockSpec(memory_space=pltpu.MemorySpace.HBM),
        ],
        out_specs=pl.BlockSpec(memory_space=pltpu.MemorySpace.VMEM),
        scratch_shapes=[
            pltpu.VMEM((D_MODEL, D_HEADS_LOCAL), jnp.float32),
            pltpu.VMEM((D_HEADS_LOCAL, D_MODEL), jnp.float32),
            pltpu.VMEM((SQ, D_MODEL), jnp.float32),
            pltpu.VMEM((SQ, D_HEADS_LOCAL), jnp.float32),
            pltpu.VMEM((SQ, D_HEADS_LOCAL), jnp.float32),
            pltpu.VMEM((SQ, D_HEADS_LOCAL), jnp.float32),
            pltpu.VMEM((SQ, D_MODEL), jnp.float32),
            pltpu.VMEM((2, CHUNK, D_MODEL), jnp.bfloat16),
            pltpu.VMEM((2, CHUNK, D_MODEL), jnp.bfloat16),
            pltpu.SemaphoreType.DMA((2,)),
            pltpu.SemaphoreType.DMA((2,)),
            pltpu.SemaphoreType.DMA((2,)),
            pltpu.SemaphoreType.DMA((2,)),
            pltpu.SemaphoreType.DMA((2,)),
        ],
        compiler_params=pltpu.CompilerParams(collective_id=0),
    )(x2, Wq, k2, v2, Wo)
    return out[None]
